# baseline (speedup 1.0000x reference)
import functools

import jax
import jax.numpy as jnp
from jax.experimental import pallas as pl
from jax.experimental.pallas import tpu as pltpu

_NEG_RATIO = 3
_DN = (((0,), (0,)), ((), ()))


def _eye(n):
    r = jax.lax.broadcasted_iota(jnp.int32, (n, n), 0)
    c = jax.lax.broadcasted_iota(jnp.int32, (n, n), 1)
    return (r == c).astype(jnp.float32)


def _mxu_t(x2d, eye128, eye_tail):
    ach = x2d.shape[0]
    n = ach // 128
    tail = ach - 128 * n
    parts = [
        jax.lax.dot_general(x2d[128 * i:128 * (i + 1), :], eye128, _DN,
                            preferred_element_type=jnp.float32)
        for i in range(n)
    ]
    if tail:
        parts.append(
            jax.lax.dot_general(x2d[128 * n:, :], eye_tail, _DN,
                                preferred_element_type=jnp.float32))
    return jnp.concatenate(parts, axis=1)


def _mbl_kernel(lab_ref, lt_ref, ploc_ref, gloc_ref, out_loc_ref, out_cls_ref,
                losses_s, np_s, acc_s, *, B, A, C, ACH, NCH):
    b = pl.program_id(0)
    j = pl.program_id(1)

    eye128 = _eye(128)
    eye_tail = _eye(ACH - 128 * (ACH // 128)) if ACH % 128 else None

    lab = lab_ref[0, 0]
    pos = lab > 0

    lt = _mxu_t(lt_ref[0, 0], eye128, eye_tail)

    m = jnp.max(lt, axis=0, keepdims=True)
    se = jnp.sum(jnp.exp(lt - m), axis=0, keepdims=True)
    lse = jnp.log(se) + m
    cls_iota = jax.lax.broadcasted_iota(jnp.int32, (C, ACH), 0)
    picked = jnp.sum(jnp.where(cls_iota == lab, lt, 0.0), axis=0, keepdims=True)
    ce = lse - picked

    losses = jnp.where(pos, 0.0, ce)
    losses_s[j, pl.ds(b, 1), :] = losses

    numpos = jnp.sum(pos.astype(jnp.int32))
    posce = jnp.sum(jnp.where(pos, ce, 0.0))

    d = (_mxu_t(ploc_ref[0, 0], eye128, eye_tail)
         - _mxu_t(gloc_ref[0, 0], eye128, eye_tail))
    ad = jnp.abs(d)
    sl1 = jnp.where(ad < 1.0, 0.5 * d * d, ad - 0.5)
    locl = jnp.sum(jnp.where(pos, sl1, 0.0))

    @pl.when(jnp.logical_and(b == 0, j == 0))
    def _init():
        acc_s[0] = 0.0
        acc_s[1] = 0.0

    @pl.when(j == 0)
    def _init_row():
        np_s[pl.ds(b, 1), :] = jnp.zeros((1, 128), jnp.int32)

    np_s[pl.ds(b, 1), :] = np_s[pl.ds(b, 1), :] + numpos
    acc_s[0] = acc_s[0] + locl
    acc_s[1] = acc_s[1] + posce

    @pl.when(jnp.logical_and(b == B - 1, j == NCH - 1))
    def _finish():
        allb = losses_s[:, :, :]
        bits = jax.lax.bitcast_convert_type(allb, jnp.int32)
        npvec = np_s[:, 0:1]
        k = jnp.minimum(_NEG_RATIO * npvec, A - 1)

        lo = jnp.zeros((B, 1), jnp.int32)
        hi = jnp.max(jnp.max(bits, axis=2), axis=0)[:, None]

        def body(_, carry):
            lo, hi = carry
            mid = lo + (hi - lo + 1) // 2
            cnt = jnp.sum(jnp.sum(
                (bits >= mid[None]).astype(jnp.int32), axis=2), axis=0)[:, None]
            ge = cnt >= k
            return jnp.where(ge, mid, lo), jnp.where(ge, hi, mid - 1)

        lo, hi = jax.lax.fori_loop(0, 31, body, (lo, hi))
        tv = jax.lax.bitcast_convert_type(lo, jnp.float32)
        gt = bits > lo[None]
        cnt_gt = jnp.sum(jnp.sum(gt.astype(jnp.int32), axis=2), axis=0)[:, None]
        sum_gt = jnp.sum(jnp.sum(
            jnp.where(gt, allb, 0.0), axis=2), axis=0)[:, None]
        topk = sum_gt + (k - cnt_gt).astype(jnp.float32) * tv
        topk = jnp.where(k >= 1, topk, 0.0)

        n = jnp.sum(npvec).astype(jnp.float32)
        out_loc_ref[:, :] = jnp.reshape(acc_s[0] / n, (1, 1))
        out_cls_ref[:, :] = jnp.reshape((acc_s[1] + jnp.sum(topk)) / n, (1, 1))


def kernel(pred_loc, pred_label, gt_loc, gt_label):
    B, A, C = pred_label.shape
    ACH = 2000
    NCH = A // ACH
    labR = gt_label.reshape(B, NCH, 1, ACH)
    ltR = pred_label.reshape(B, NCH, ACH, C)
    plocR = pred_loc.reshape(B, NCH, ACH, 4)
    glocR = gt_loc.reshape(B, NCH, ACH, 4)

    grid = (B, NCH)
    out_loc, out_cls = pl.pallas_call(
        functools.partial(_mbl_kernel, B=B, A=A, C=C, ACH=ACH, NCH=NCH),
        grid=grid,
        in_specs=[
            pl.BlockSpec((1, 1, 1, ACH), lambda b, j: (b, j, 0, 0)),
            pl.BlockSpec((1, 1, ACH, C), lambda b, j: (b, j, 0, 0)),
            pl.BlockSpec((1, 1, ACH, 4), lambda b, j: (b, j, 0, 0)),
            pl.BlockSpec((1, 1, ACH, 4), lambda b, j: (b, j, 0, 0)),
        ],
        out_specs=[
            pl.BlockSpec((1, 1), lambda b, j: (0, 0)),
            pl.BlockSpec((1, 1), lambda b, j: (0, 0)),
        ],
        out_shape=[
            jax.ShapeDtypeStruct((1, 1), jnp.float32),
            jax.ShapeDtypeStruct((1, 1), jnp.float32),
        ],
        scratch_shapes=[
            pltpu.VMEM((NCH, B, ACH), jnp.float32),
            pltpu.VMEM((B, 128), jnp.int32),
            pltpu.SMEM((2,), jnp.float32),
        ],
    )(labR, ltR, plocR, glocR)
    return (out_loc.reshape(()), out_cls.reshape(()))


# speedup vs baseline: 11.6849x; 11.6849x over previous
import functools

import jax
import jax.numpy as jnp
from jax.experimental import pallas as pl
from jax.experimental.pallas import tpu as pltpu

_NEG_RATIO = 3


def _phase1_kernel(lab_ref, lt_ref, ploc_ref, gloc_ref,
                   losses_ref, np_ref, locsum_ref, posce_ref, acc_s, *, G, C):
    b = pl.program_id(0)

    lab = lab_ref[0]
    lt = lt_ref[0]
    pos = lab > 0

    m = jnp.max(lt, axis=0, keepdims=True)
    se = jnp.sum(jnp.exp(lt - m), axis=0, keepdims=True)
    lse = jnp.log(se) + m
    cls_iota = jax.lax.broadcasted_iota(jnp.int32, lt.shape, 0)
    picked = jnp.sum(jnp.where(cls_iota == lab, lt, 0.0), axis=0, keepdims=True)
    ce = lse - picked

    losses_ref[0] = jnp.where(pos, 0.0, ce)

    numpos = jnp.sum(pos.astype(jnp.int32))
    posce = jnp.sum(jnp.where(pos, ce, 0.0))

    d = ploc_ref[0] - gloc_ref[0]
    ad = jnp.abs(d)
    sl1 = jnp.where(ad < 1.0, 0.5 * d * d, ad - 0.5)
    locl = jnp.sum(jnp.where(pos, sl1, 0.0))

    np_ref[0] = jnp.broadcast_to(numpos, (1, 128))

    @pl.when(b == 0)
    def _init():
        acc_s[0] = 0.0
        acc_s[1] = 0.0

    acc_s[0] = acc_s[0] + locl
    acc_s[1] = acc_s[1] + posce

    @pl.when(b == G - 1)
    def _finish():
        locsum_ref[:, :] = jnp.reshape(acc_s[0], (1, 1))
        posce_ref[:, :] = jnp.reshape(acc_s[1], (1, 1))


def _phase1(pred_loc, pred_label, gt_loc, gt_label):
    G, A, C = pred_label.shape
    labT = gt_label.reshape(G, 1, A)
    ltT = pred_label.transpose(0, 2, 1)
    plocT = pred_loc.transpose(0, 2, 1)
    glocT = gt_loc.transpose(0, 2, 1)

    return pl.pallas_call(
        functools.partial(_phase1_kernel, G=G, C=C),
        grid=(G,),
        in_specs=[
            pl.BlockSpec((1, 1, A), lambda b: (b, 0, 0)),
            pl.BlockSpec((1, C, A), lambda b: (b, 0, 0)),
            pl.BlockSpec((1, 4, A), lambda b: (b, 0, 0)),
            pl.BlockSpec((1, 4, A), lambda b: (b, 0, 0)),
        ],
        out_specs=[
            pl.BlockSpec((1, 1, A), lambda b: (b, 0, 0)),
            pl.BlockSpec((1, 1, 128), lambda b: (b, 0, 0)),
            pl.BlockSpec((1, 1), lambda b: (0, 0)),
            pl.BlockSpec((1, 1), lambda b: (0, 0)),
        ],
        out_shape=[
            jax.ShapeDtypeStruct((G, 1, A), jnp.float32),
            jax.ShapeDtypeStruct((G, 1, 128), jnp.int32),
            jax.ShapeDtypeStruct((1, 1), jnp.float32),
            jax.ShapeDtypeStruct((1, 1), jnp.float32),
        ],
        scratch_shapes=[pltpu.SMEM((2,), jnp.float32)],
    )(labT, ltT, plocT, glocT)


def _phase2_kernel(*refs, B, A, NG):
    losses_refs = refs[:NG]
    np_refs = refs[NG:2 * NG]
    out_ref = refs[2 * NG]

    allb = jnp.concatenate([r[:, 0, :] for r in losses_refs], axis=0)
    bits = jax.lax.bitcast_convert_type(allb, jnp.int32)
    npvec = jnp.concatenate([r[:, 0, 0:1] for r in np_refs], axis=0)
    k = jnp.minimum(_NEG_RATIO * npvec, A - 1)

    lo = jnp.zeros((B, 1), jnp.int32)
    hi = jnp.max(bits, axis=1, keepdims=True)

    def body(_, carry):
        lo, hi = carry
        mid = lo + (hi - lo + 1) // 2
        cnt = jnp.sum((bits >= mid).astype(jnp.int32), axis=1, keepdims=True)
        ge = cnt >= k
        return jnp.where(ge, mid, lo), jnp.where(ge, hi, mid - 1)

    lo, hi = jax.lax.fori_loop(0, 31, body, (lo, hi))
    tv = jax.lax.bitcast_convert_type(lo, jnp.float32)
    gt = bits > lo
    cnt_gt = jnp.sum(gt.astype(jnp.int32), axis=1, keepdims=True)
    sum_gt = jnp.sum(jnp.where(gt, allb, 0.0), axis=1, keepdims=True)
    topk = sum_gt + (k - cnt_gt).astype(jnp.float32) * tv
    topk = jnp.where(k >= 1, topk, 0.0)
    out_ref[:, :] = jnp.reshape(jnp.sum(topk), (1, 1))


def kernel(pred_loc, pred_label, gt_loc, gt_label):
    B, A, C = pred_label.shape
    NG = 4
    G = B // NG

    losses, nps, locsums, posces = [], [], [], []
    for g in range(NG):
        sl = slice(g * G, (g + 1) * G)
        lo_g, np_g, locsum_g, posce_g = _phase1(
            pred_loc[sl], pred_label[sl], gt_loc[sl], gt_label[sl])
        losses.append(lo_g)
        nps.append(np_g)
        locsums.append(locsum_g)
        posces.append(posce_g)

    topk_sum = pl.pallas_call(
        functools.partial(_phase2_kernel, B=B, A=A, NG=NG),
        out_shape=jax.ShapeDtypeStruct((1, 1), jnp.float32),
    )(*losses, *nps)

    n = sum(jnp.sum(np_g[:, 0, 0]) for np_g in nps).astype(jnp.float32)
    loc_loss = sum(x.reshape(()) for x in locsums) / n
    cls_loss = (sum(x.reshape(()) for x in posces) + topk_sum.reshape(())) / n
    return (loc_loss, cls_loss)
